# baseline (speedup 1.0000x reference)
import jax
import jax.numpy as jnp
from jax.experimental import pallas as pl
from jax.experimental.pallas import tpu as pltpu

_UV = 10000
_DIN = 300
_HID = 64
_DOUT = 64
_B = 4096

_T = 200
_G = 256


def _z1_body(embed_ref, w1_ref, b1_ref, o_ref):
    o_ref[...] = (
        jnp.dot(embed_ref[...], w1_ref[...], preferred_element_type=jnp.float32)
        + b1_ref[...]
    )


def _z2_body(adj_ref, z1_ref, w2_ref, o_ref):
    a = adj_ref[...].astype(jnp.float32)
    h = jnp.dot(a, z1_ref[...], preferred_element_type=jnp.float32)
    h = jnp.maximum(h, 0.0)
    o_ref[...] = jnp.dot(h, w2_ref[...], preferred_element_type=jnp.float32)


def _out_body(tid_ref, adj_ref, z2_ref, b2_ref, o_ref, buf, sems):
    i = pl.program_id(0)
    ng = _B // _G

    def _issue(slot, grp):
        for g in range(_G):
            pltpu.make_async_copy(
                adj_ref.at[pl.ds(tid_ref[grp * _G + g], 1), :],
                buf.at[slot, pl.ds(g, 1), :],
                sems.at[slot],
            ).start()

    @pl.when(i == 0)
    def _():
        _issue(0, 0)

    @pl.when(i + 1 < ng)
    def _():
        _issue((i + 1) % 2, i + 1)

    slot = i % 2
    pltpu.make_async_copy(
        adj_ref.at[pl.ds(0, _G), :], buf.at[slot], sems.at[slot]
    ).wait()

    a = buf[slot].astype(jnp.float32)
    o_ref[...] = (
        jnp.dot(a, z2_ref[...], preferred_element_type=jnp.float32) + b2_ref[...]
    )


def kernel(X_tid, adj, embed, W1, b1, W2, b2):
    b1r = jnp.reshape(b1, (1, _HID))
    b2r = jnp.reshape(b2, (1, _DOUT))

    z1b = pl.pallas_call(
        _z1_body,
        out_shape=jax.ShapeDtypeStruct((_UV, _HID), jnp.float32),
        in_specs=[
            pl.BlockSpec((_UV, _DIN), lambda: (0, 0)),
            pl.BlockSpec((_DIN, _HID), lambda: (0, 0)),
            pl.BlockSpec((1, _HID), lambda: (0, 0)),
        ],
        out_specs=pl.BlockSpec((_UV, _HID), lambda: (0, 0)),
    )(embed, W1, b1r)

    nt = _UV // _T
    z2 = pl.pallas_call(
        _z2_body,
        grid=(nt,),
        out_shape=jax.ShapeDtypeStruct((_UV, _DOUT), jnp.float32),
        in_specs=[
            pl.BlockSpec((_T, _UV), lambda i: (i, 0)),
            pl.BlockSpec((_UV, _HID), lambda i: (0, 0)),
            pl.BlockSpec((_HID, _DOUT), lambda i: (0, 0)),
        ],
        out_specs=pl.BlockSpec((_T, _DOUT), lambda i: (i, 0)),
    )(adj, z1b, W2)

    ng = _B // _G
    out = pl.pallas_call(
        _out_body,
        grid_spec=pltpu.PrefetchScalarGridSpec(
            num_scalar_prefetch=1,
            grid=(ng,),
            in_specs=[
                pl.BlockSpec(memory_space=pl.ANY),
                pl.BlockSpec((_UV, _DOUT), lambda i, tid: (0, 0)),
                pl.BlockSpec((1, _DOUT), lambda i, tid: (0, 0)),
            ],
            out_specs=pl.BlockSpec((_G, _DOUT), lambda i, tid: (i, 0)),
            scratch_shapes=[
                pltpu.VMEM((2, _G, _UV), jnp.int32),
                pltpu.SemaphoreType.DMA((2,)),
            ],
        ),
        out_shape=jax.ShapeDtypeStruct((_B, _DOUT), jnp.float32),
    )(X_tid, adj, z2, b2r)
    return out


# speedup vs baseline: 1.0104x; 1.0104x over previous
import jax
import jax.numpy as jnp
from jax.experimental import pallas as pl
from jax.experimental.pallas import tpu as pltpu

_UV = 10000
_DIN = 300
_HID = 64
_DOUT = 64
_B = 4096

_T = 400
_G = 512


def _z1_body(embed_ref, w1_ref, b1_ref, o_ref):
    o_ref[...] = (
        jnp.dot(embed_ref[...], w1_ref[...], preferred_element_type=jnp.float32)
        + b1_ref[...]
    )


def _z2_body(adj_ref, z1_ref, w2_ref, o_ref):
    a = adj_ref[...].astype(jnp.float32)
    h = jnp.dot(a, z1_ref[...], preferred_element_type=jnp.float32)
    h = jnp.maximum(h, 0.0)
    o_ref[...] = jnp.dot(h, w2_ref[...], preferred_element_type=jnp.float32)


def _out_body(tid_ref, adj_ref, z2_ref, b2_ref, o_ref, buf, sems):
    i = pl.program_id(0)
    ng = _B // _G

    def _issue(slot, grp):
        for g in range(_G):
            pltpu.make_async_copy(
                adj_ref.at[pl.ds(tid_ref[grp * _G + g], 1), :],
                buf.at[slot, pl.ds(g, 1), :],
                sems.at[slot],
            ).start()

    @pl.when(i == 0)
    def _():
        _issue(0, 0)

    @pl.when(i + 1 < ng)
    def _():
        _issue((i + 1) % 2, i + 1)

    slot = i % 2
    pltpu.make_async_copy(
        adj_ref.at[pl.ds(0, _G), :], buf.at[slot], sems.at[slot]
    ).wait()

    a = buf[slot].astype(jnp.float32)
    o_ref[...] = (
        jnp.dot(a, z2_ref[...], preferred_element_type=jnp.float32) + b2_ref[...]
    )


def kernel(X_tid, adj, embed, W1, b1, W2, b2):
    b1r = jnp.reshape(b1, (1, _HID))
    b2r = jnp.reshape(b2, (1, _DOUT))

    z1b = pl.pallas_call(
        _z1_body,
        out_shape=jax.ShapeDtypeStruct((_UV, _HID), jnp.float32),
        in_specs=[
            pl.BlockSpec((_UV, _DIN), lambda: (0, 0)),
            pl.BlockSpec((_DIN, _HID), lambda: (0, 0)),
            pl.BlockSpec((1, _HID), lambda: (0, 0)),
        ],
        out_specs=pl.BlockSpec((_UV, _HID), lambda: (0, 0)),
    )(embed, W1, b1r)

    nt = _UV // _T
    z2 = pl.pallas_call(
        _z2_body,
        grid=(nt,),
        out_shape=jax.ShapeDtypeStruct((_UV, _DOUT), jnp.float32),
        in_specs=[
            pl.BlockSpec((_T, _UV), lambda i: (i, 0)),
            pl.BlockSpec((_UV, _HID), lambda i: (0, 0)),
            pl.BlockSpec((_HID, _DOUT), lambda i: (0, 0)),
        ],
        out_specs=pl.BlockSpec((_T, _DOUT), lambda i: (i, 0)),
    )(adj, z1b, W2)

    ng = _B // _G
    out = pl.pallas_call(
        _out_body,
        grid_spec=pltpu.PrefetchScalarGridSpec(
            num_scalar_prefetch=1,
            grid=(ng,),
            in_specs=[
                pl.BlockSpec(memory_space=pl.ANY),
                pl.BlockSpec((_UV, _DOUT), lambda i, tid: (0, 0)),
                pl.BlockSpec((1, _DOUT), lambda i, tid: (0, 0)),
            ],
            out_specs=pl.BlockSpec((_G, _DOUT), lambda i, tid: (i, 0)),
            scratch_shapes=[
                pltpu.VMEM((2, _G, _UV), jnp.int32),
                pltpu.SemaphoreType.DMA((2,)),
            ],
        ),
        out_shape=jax.ShapeDtypeStruct((_B, _DOUT), jnp.float32),
    )(X_tid, adj, z2, b2r)
    return out


# speedup vs baseline: 1.0235x; 1.0130x over previous
import jax
import jax.numpy as jnp
from jax.experimental import pallas as pl
from jax.experimental.pallas import tpu as pltpu

_UV = 10000
_DIN = 300
_HID = 64
_DOUT = 64
_B = 4096

_T = 200
_NT = _UV // _T
_G = 128
_NG = _B // _G
_S = 4


def _z1_body(embed_ref, w1_ref, b1_ref, o_ref):
    o_ref[...] = (
        jnp.dot(embed_ref[...], w1_ref[...], preferred_element_type=jnp.float32)
        + b1_ref[...]
    )


def _fused_body(tid_ref, adj_blk, z1_ref, w2_ref, b2_ref, adj_any, o_ref,
                z2_s, buf, sems):
    i = pl.program_id(0)

    def _issue(grp):
        slot = jax.lax.rem(grp, _S)
        for g in range(_G):
            pltpu.make_async_copy(
                adj_any.at[pl.ds(tid_ref[grp * _G + g], 1), :],
                buf.at[slot, pl.ds(g, 1), :],
                sems.at[slot],
            ).start()

    @pl.when(i < _NT)
    def _():
        a = adj_blk[...].astype(jnp.float32)
        h = jnp.dot(a, z1_ref[...], preferred_element_type=jnp.float32)
        h = jnp.maximum(h, 0.0)
        z2_s[pl.ds(i * _T, _T), :] = jnp.dot(
            h, w2_ref[...], preferred_element_type=jnp.float32
        )

    @pl.when(i == _NT - 1)
    def _():
        for grp in range(min(_S - 1, _NG)):
            _issue(grp)

    @pl.when(i >= _NT)
    def _():
        j = i - _NT

        @pl.when(j + _S - 1 < _NG)
        def _():
            _issue(j + _S - 1)

        slot = jax.lax.rem(j, _S)
        pltpu.make_async_copy(
            adj_any.at[pl.ds(0, _G), :], buf.at[slot], sems.at[slot]
        ).wait()
        a = buf[slot].astype(jnp.float32)
        o_ref[...] = (
            jnp.dot(a, z2_s[...], preferred_element_type=jnp.float32)
            + b2_ref[...]
        )


def kernel(X_tid, adj, embed, W1, b1, W2, b2):
    b1r = jnp.reshape(b1, (1, _HID))
    b2r = jnp.reshape(b2, (1, _DOUT))

    z1b = pl.pallas_call(
        _z1_body,
        out_shape=jax.ShapeDtypeStruct((_UV, _HID), jnp.float32),
        in_specs=[
            pl.BlockSpec((_UV, _DIN), lambda: (0, 0)),
            pl.BlockSpec((_DIN, _HID), lambda: (0, 0)),
            pl.BlockSpec((1, _HID), lambda: (0, 0)),
        ],
        out_specs=pl.BlockSpec((_UV, _HID), lambda: (0, 0)),
    )(embed, W1, b1r)

    out = pl.pallas_call(
        _fused_body,
        grid_spec=pltpu.PrefetchScalarGridSpec(
            num_scalar_prefetch=1,
            grid=(_NT + _NG,),
            in_specs=[
                pl.BlockSpec((_T, _UV), lambda i, tid: (jnp.minimum(i, _NT - 1), 0)),
                pl.BlockSpec((_UV, _HID), lambda i, tid: (0, 0)),
                pl.BlockSpec((_HID, _DOUT), lambda i, tid: (0, 0)),
                pl.BlockSpec((1, _DOUT), lambda i, tid: (0, 0)),
                pl.BlockSpec(memory_space=pl.ANY),
            ],
            out_specs=pl.BlockSpec(
                (_G, _DOUT), lambda i, tid: (jnp.maximum(i - _NT, 0), 0)
            ),
            scratch_shapes=[
                pltpu.VMEM((_UV, _DOUT), jnp.float32),
                pltpu.VMEM((_S, _G, _UV), jnp.int32),
                pltpu.SemaphoreType.DMA((_S,)),
            ],
        ),
        out_shape=jax.ShapeDtypeStruct((_B, _DOUT), jnp.float32),
    )(X_tid, adj, z1b, W2, b2r, adj)
    return out


# speedup vs baseline: 1.0476x; 1.0235x over previous
import jax
import jax.numpy as jnp
from jax.experimental import pallas as pl
from jax.experimental.pallas import tpu as pltpu

_UV = 10000
_DIN = 300
_HID = 64
_DOUT = 64
_B = 4096

_T = 200
_NT = _UV // _T
_ST = 3
_G = 128
_NG = _B // _G
_S = 4


def _z1_body(embed_ref, w1_ref, b1_ref, o_ref):
    o_ref[...] = (
        jnp.dot(embed_ref[...], w1_ref[...], preferred_element_type=jnp.float32)
        + b1_ref[...]
    )


def _fused_body(tid_ref, z1_ref, w2_ref, b1_ref, b2_ref, adj_any, o_ref,
                z2_s, tbuf, buf, tsems, sems):
    i = pl.program_id(0)

    def _issue_tile(t):
        pltpu.make_async_copy(
            adj_any.at[pl.ds(t * _T, _T), :],
            tbuf.at[jax.lax.rem(t, _ST)],
            tsems.at[jax.lax.rem(t, _ST)],
        ).start()

    def _issue_rows(grp):
        slot = jax.lax.rem(grp, _S)
        for g in range(_G):
            pltpu.make_async_copy(
                adj_any.at[pl.ds(tid_ref[grp * _G + g], 1), :],
                buf.at[slot, pl.ds(g, 1), :],
                sems.at[slot],
            ).start()

    @pl.when(i == 0)
    def _():
        for t in range(_ST - 1):
            _issue_tile(t)

    @pl.when(i < _NT)
    def _():
        @pl.when(i + _ST - 1 < _NT)
        def _():
            _issue_tile(i + _ST - 1)

        slot = jax.lax.rem(i, _ST)
        pltpu.make_async_copy(
            adj_any.at[pl.ds(0, _T), :], tbuf.at[slot], tsems.at[slot]
        ).wait()
        a = tbuf[slot].astype(jnp.float32)
        h = jnp.dot(a, z1_ref[...], preferred_element_type=jnp.float32)
        h = jnp.maximum(h + b1_ref[...], 0.0)
        z2_s[pl.ds(i * _T, _T), :] = jnp.dot(
            h, w2_ref[...], preferred_element_type=jnp.float32
        )

    @pl.when(i == _NT - 1)
    def _():
        for grp in range(min(_S - 1, _NG)):
            _issue_rows(grp)

    @pl.when(i >= _NT)
    def _():
        j = i - _NT

        @pl.when(j + _S - 1 < _NG)
        def _():
            _issue_rows(j + _S - 1)

        slot = jax.lax.rem(j, _S)
        pltpu.make_async_copy(
            adj_any.at[pl.ds(0, _G), :], buf.at[slot], sems.at[slot]
        ).wait()
        a = buf[slot].astype(jnp.float32)
        o_ref[...] = (
            jnp.dot(a, z2_s[...], preferred_element_type=jnp.float32)
            + b2_ref[...]
        )


def kernel(X_tid, adj, embed, W1, b1, W2, b2):
    b1r = jnp.reshape(b1, (1, _HID))
    b2r = jnp.reshape(b2, (1, _DOUT))

    z1b = pl.pallas_call(
        _z1_body,
        out_shape=jax.ShapeDtypeStruct((_UV, _HID), jnp.float32),
        in_specs=[
            pl.BlockSpec((_UV, _DIN), lambda: (0, 0)),
            pl.BlockSpec((_DIN, _HID), lambda: (0, 0)),
            pl.BlockSpec((1, _HID), lambda: (0, 0)),
        ],
        out_specs=pl.BlockSpec((_UV, _HID), lambda: (0, 0)),
    )(embed, W1, b1r)

    out = pl.pallas_call(
        _fused_body,
        grid_spec=pltpu.PrefetchScalarGridSpec(
            num_scalar_prefetch=1,
            grid=(_NT + _NG,),
            in_specs=[
                pl.BlockSpec((_UV, _HID), lambda i, tid: (0, 0)),
                pl.BlockSpec((_HID, _DOUT), lambda i, tid: (0, 0)),
                pl.BlockSpec((1, _HID), lambda i, tid: (0, 0)),
                pl.BlockSpec((1, _DOUT), lambda i, tid: (0, 0)),
                pl.BlockSpec(memory_space=pl.ANY),
            ],
            out_specs=pl.BlockSpec(
                (_G, _DOUT), lambda i, tid: (jnp.maximum(i - _NT, 0), 0)
            ),
            scratch_shapes=[
                pltpu.VMEM((_UV, _DOUT), jnp.float32),
                pltpu.VMEM((_ST, _T, _UV), jnp.int32),
                pltpu.VMEM((_S, _G, _UV), jnp.int32),
                pltpu.SemaphoreType.DMA((_ST,)),
                pltpu.SemaphoreType.DMA((_S,)),
            ],
        ),
        out_shape=jax.ShapeDtypeStruct((_B, _DOUT), jnp.float32),
    )(X_tid, z1b, W2, b1r, b2r, adj)
    return out
